# baseline (speedup 1.0000x reference)
import jax
import jax.numpy as jnp
from jax import lax
from jax.experimental import pallas as pl
from jax.experimental.pallas import tpu as pltpu

_G = 8
_CHUNK = 16


def _index_kernel(mask_ref, idx_ref):
    B, L = mask_ref.shape
    pos = lax.broadcasted_iota(jnp.int32, (B, L), 1)
    m = mask_ref[...].astype(jnp.int32)
    cand = jnp.where(m == 0, pos, -1)
    idx = jnp.max(cand, axis=1)
    idx_ref[...] = jnp.maximum(idx, 0)


def _gather_kernel(idx_ref, vals_ref, out_ref, sem):
    B, L, D = vals_ref.shape

    def group(g, _):
        def fire(b, _):
            pltpu.make_async_copy(
                vals_ref.at[b, idx_ref[b]], out_ref.at[b, 0], sem
            ).start()
            return 0

        def drain(b, _):
            pltpu.make_async_copy(
                vals_ref.at[b, 0], out_ref.at[b, 0], sem
            ).wait()
            return 0

        lax.fori_loop(g * _CHUNK, (g + 1) * _CHUNK, fire, 0)
        lax.fori_loop(g * _CHUNK, (g + 1) * _CHUNK, drain, 0)
        return 0

    lax.fori_loop(0, B // _CHUNK, group, 0)


def _broadcast_kernel(lv_ref, out_ref):
    out_ref[...] = jnp.broadcast_to(lv_ref[...], out_ref.shape)


def kernel(input_values, input_timestamps, is_target_mask, dummy):
    B, L, D = input_values.shape
    mask_i8 = is_target_mask.astype(jnp.int8)

    idx = pl.pallas_call(
        _index_kernel,
        out_shape=jax.ShapeDtypeStruct((B,), jnp.int32),
    )(mask_i8)

    last_values = pl.pallas_call(
        _gather_kernel,
        in_specs=[
            pl.BlockSpec(memory_space=pltpu.SMEM),
            pl.BlockSpec(memory_space=pl.ANY),
        ],
        out_specs=pl.BlockSpec(memory_space=pltpu.VMEM),
        scratch_shapes=[pltpu.SemaphoreType.DMA],
        out_shape=jax.ShapeDtypeStruct((B, 1, D), jnp.float32),
    )(idx, input_values)

    out = pl.pallas_call(
        _broadcast_kernel,
        grid=(B // _G,),
        in_specs=[pl.BlockSpec((_G, 1, D), lambda i: (i, 0, 0))],
        out_specs=pl.BlockSpec((_G, L, D), lambda i: (i, 0, 0)),
        out_shape=jax.ShapeDtypeStruct((B, L, D), jnp.float32),
    )(last_values)
    return out


# speedup vs baseline: 1.1046x; 1.1046x over previous
import jax
import jax.numpy as jnp
from jax import lax
from jax.experimental import pallas as pl
from jax.experimental.pallas import tpu as pltpu

_G = 4
_CHUNK = 64


def _index_kernel(mask_ref, idx_ref):
    B, L = mask_ref.shape
    pos = lax.broadcasted_iota(jnp.int32, (B, L), 1)
    m = mask_ref[...].astype(jnp.int32)
    cand = jnp.where(m == 0, pos, -1)
    idx = jnp.max(cand, axis=1)
    idx_ref[...] = jnp.maximum(idx, 0)


def _gather_kernel(idx_ref, vals_ref, out_ref, sem):
    B, L, D = vals_ref.shape

    def group(g, _):
        def fire(b, _):
            pltpu.make_async_copy(
                vals_ref.at[b, idx_ref[b]], out_ref.at[b, 0], sem
            ).start()
            return 0

        def drain(b, _):
            pltpu.make_async_copy(
                vals_ref.at[b, 0], out_ref.at[b, 0], sem
            ).wait()
            return 0

        lax.fori_loop(g * _CHUNK, (g + 1) * _CHUNK, fire, 0)
        lax.fori_loop(g * _CHUNK, (g + 1) * _CHUNK, drain, 0)
        return 0

    lax.fori_loop(0, B // _CHUNK, group, 0)


def _broadcast_kernel(lv_ref, out_ref):
    out_ref[...] = jnp.broadcast_to(lv_ref[...], out_ref.shape)


def kernel(input_values, input_timestamps, is_target_mask, dummy):
    B, L, D = input_values.shape
    mask_i8 = is_target_mask.astype(jnp.int8)

    idx = pl.pallas_call(
        _index_kernel,
        out_shape=jax.ShapeDtypeStruct((B,), jnp.int32),
    )(mask_i8)

    last_values = pl.pallas_call(
        _gather_kernel,
        in_specs=[
            pl.BlockSpec(memory_space=pltpu.SMEM),
            pl.BlockSpec(memory_space=pl.ANY),
        ],
        out_specs=pl.BlockSpec(memory_space=pltpu.VMEM),
        scratch_shapes=[pltpu.SemaphoreType.DMA],
        out_shape=jax.ShapeDtypeStruct((B, 1, D), jnp.float32),
    )(idx, input_values)

    out = pl.pallas_call(
        _broadcast_kernel,
        grid=(B // _G,),
        in_specs=[pl.BlockSpec((_G, 1, D), lambda i: (i, 0, 0))],
        out_specs=pl.BlockSpec((_G, L, D), lambda i: (i, 0, 0)),
        out_shape=jax.ShapeDtypeStruct((B, L, D), jnp.float32),
    )(last_values)
    return out


# speedup vs baseline: 1.1158x; 1.0102x over previous
import jax
import jax.numpy as jnp
from jax import lax
from jax.experimental import pallas as pl
from jax.experimental.pallas import tpu as pltpu

_G = 4
_CHUNK = 256


def _index_kernel(mask_ref, idx_ref):
    B, L = mask_ref.shape
    pos = lax.broadcasted_iota(jnp.int32, (B, L), 1)
    m = mask_ref[...].astype(jnp.int32)
    cand = jnp.where(m == 0, pos, -1)
    idx = jnp.max(cand, axis=1)
    idx_ref[...] = jnp.maximum(idx, 0)


def _gather_kernel(idx_ref, vals_ref, out_ref, sem):
    B, L, D = vals_ref.shape

    def group(g, _):
        def fire(b, _):
            pltpu.make_async_copy(
                vals_ref.at[b, idx_ref[b]], out_ref.at[b, 0], sem
            ).start()
            return 0

        def drain(b, _):
            pltpu.make_async_copy(
                vals_ref.at[b, 0], out_ref.at[b, 0], sem
            ).wait()
            return 0

        lax.fori_loop(g * _CHUNK, (g + 1) * _CHUNK, fire, 0)
        lax.fori_loop(g * _CHUNK, (g + 1) * _CHUNK, drain, 0)
        return 0

    lax.fori_loop(0, B // _CHUNK, group, 0)


def _broadcast_kernel(lv_ref, out_ref):
    out_ref[...] = jnp.broadcast_to(lv_ref[...], out_ref.shape)


def kernel(input_values, input_timestamps, is_target_mask, dummy):
    B, L, D = input_values.shape
    mask_i8 = is_target_mask.astype(jnp.int8)

    idx = pl.pallas_call(
        _index_kernel,
        out_shape=jax.ShapeDtypeStruct((B,), jnp.int32),
    )(mask_i8)

    last_values = pl.pallas_call(
        _gather_kernel,
        in_specs=[
            pl.BlockSpec(memory_space=pltpu.SMEM),
            pl.BlockSpec(memory_space=pl.ANY),
        ],
        out_specs=pl.BlockSpec(memory_space=pltpu.VMEM),
        scratch_shapes=[pltpu.SemaphoreType.DMA],
        out_shape=jax.ShapeDtypeStruct((B, 1, D), jnp.float32),
    )(idx, input_values)

    out = pl.pallas_call(
        _broadcast_kernel,
        grid=(B // _G,),
        in_specs=[pl.BlockSpec((_G, 1, D), lambda i: (i, 0, 0))],
        out_specs=pl.BlockSpec((_G, L, D), lambda i: (i, 0, 0)),
        out_shape=jax.ShapeDtypeStruct((B, L, D), jnp.float32),
    )(last_values)
    return out


# speedup vs baseline: 1.1259x; 1.0091x over previous
import jax
import jax.numpy as jnp
from jax import lax
from jax.experimental import pallas as pl
from jax.experimental.pallas import tpu as pltpu

_G = 4


def _gather_kernel(mask_ref, vals_ref, out_ref, idx_vmem, idx_smem, sem):
    B, L = mask_ref.shape
    pos = lax.broadcasted_iota(jnp.int32, (B, L), 1)
    m = mask_ref[...].astype(jnp.int32)
    cand = jnp.where(m == 0, pos, -1)
    idx_vmem[...] = jnp.maximum(jnp.max(cand, axis=1), 0)
    cp = pltpu.make_async_copy(idx_vmem, idx_smem, sem)
    cp.start()
    cp.wait()

    def fire(b, _):
        pltpu.make_async_copy(
            vals_ref.at[b, idx_smem[b]], out_ref.at[b, 0], sem
        ).start()
        return 0

    def drain(b, _):
        pltpu.make_async_copy(vals_ref.at[b, 0], out_ref.at[b, 0], sem).wait()
        return 0

    lax.fori_loop(0, B, fire, 0)
    lax.fori_loop(0, B, drain, 0)


def _broadcast_kernel(lv_ref, out_ref):
    out_ref[...] = jnp.broadcast_to(lv_ref[...], out_ref.shape)


def kernel(input_values, input_timestamps, is_target_mask, dummy):
    B, L, D = input_values.shape
    mask_i8 = is_target_mask.astype(jnp.int8)

    last_values = pl.pallas_call(
        _gather_kernel,
        in_specs=[
            pl.BlockSpec(memory_space=pltpu.VMEM),
            pl.BlockSpec(memory_space=pl.ANY),
        ],
        out_specs=pl.BlockSpec(memory_space=pltpu.VMEM),
        scratch_shapes=[
            pltpu.VMEM((B,), jnp.int32),
            pltpu.SMEM((B,), jnp.int32),
            pltpu.SemaphoreType.DMA,
        ],
        out_shape=jax.ShapeDtypeStruct((B, 1, D), jnp.float32),
    )(mask_i8, input_values)

    out = pl.pallas_call(
        _broadcast_kernel,
        grid=(B // _G,),
        in_specs=[pl.BlockSpec((_G, 1, D), lambda i: (i, 0, 0))],
        out_specs=pl.BlockSpec((_G, L, D), lambda i: (i, 0, 0)),
        out_shape=jax.ShapeDtypeStruct((B, L, D), jnp.float32),
    )(last_values)
    return out
